# baseline (speedup 1.0000x reference)
import functools
import math

import jax
import jax.numpy as jnp
from jax import lax
from jax.experimental import pallas as pl
from jax.experimental.pallas import tpu as pltpu

WORD_NUM = 50000
QUERY_NUM = 10000
ENTITY_NUM = 50000
REVIEW_NUM = 25000
DW = 128
DE = 128
HEADS = 4
CONV = 2
LQ = 8
LR = 16
EP = 200000
B = 1024
DH = DW // HEADS

BLK_ROWS = 1024
GRP = 128


def _doc_embed_body(L, x_ref, wq_ref, wk_ref, wv_ref, o_ref, o_scratch):
    x = x_ref[...]
    q = jnp.dot(x, wq_ref[...], preferred_element_type=jnp.float32)
    k = jnp.dot(x, wk_ref[...], preferred_element_type=jnp.float32)
    v = jnp.dot(x, wv_ref[...], preferred_element_type=jnp.float32)
    scale = 1.0 / math.sqrt(DH)
    ri = lax.broadcasted_iota(jnp.int32, (GRP, GRP), 0) // L
    ci = lax.broadcasted_iota(jnp.int32, (GRP, GRP), 1) // L
    mask = ri == ci
    n_grp = BLK_ROWS // GRP
    for g in range(n_grp):
        qg = q[g * GRP:(g + 1) * GRP, :]
        kg = k[g * GRP:(g + 1) * GRP, :]
        vg = v[g * GRP:(g + 1) * GRP, :]
        for h in range(HEADS):
            qh = qg[:, h * DH:(h + 1) * DH] * scale
            kh = kg[:, h * DH:(h + 1) * DH]
            vh = vg[:, h * DH:(h + 1) * DH]
            s = lax.dot_general(qh, kh, (((1,), (1,)), ((), ())),
                                preferred_element_type=jnp.float32)
            s = jnp.where(mask, s, -1e30)
            s = s - jnp.max(s, axis=-1, keepdims=True)
            p = jnp.exp(s)
            p = p / jnp.sum(p, axis=-1, keepdims=True)
            oh = jnp.dot(p, vh, preferred_element_type=jnp.float32)
            o_scratch[g * GRP:(g + 1) * GRP, h * DH:(h + 1) * DH] = oh
    n_docs = BLK_ROWS // L
    pr = lax.broadcasted_iota(jnp.int32, (n_docs, BLK_ROWS), 0)
    pc = lax.broadcasted_iota(jnp.int32, (n_docs, BLK_ROWS), 1) // L
    pool = jnp.where(pr == pc, jnp.float32(1.0 / L), jnp.float32(0.0))
    o_ref[...] = jnp.dot(pool, o_scratch[...], preferred_element_type=jnp.float32)


def _doc_embed(xrows, Wq, Wk, Wv, L):
    rows = xrows.shape[0]
    assert rows % BLK_ROWS == 0
    n_blocks = rows // BLK_ROWS
    docs_per_blk = BLK_ROWS // L
    out = pl.pallas_call(
        functools.partial(_doc_embed_body, L),
        grid=(n_blocks,),
        in_specs=[
            pl.BlockSpec((BLK_ROWS, DW), lambda i: (i, 0)),
            pl.BlockSpec((DW, DW), lambda i: (0, 0)),
            pl.BlockSpec((DW, DW), lambda i: (0, 0)),
            pl.BlockSpec((DW, DW), lambda i: (0, 0)),
        ],
        out_specs=pl.BlockSpec((docs_per_blk, DW), lambda i: (i, 0)),
        out_shape=jax.ShapeDtypeStruct((n_blocks * docs_per_blk, DW), jnp.float32),
        scratch_shapes=[pltpu.VMEM((BLK_ROWS, DW), jnp.float32)],
    )(xrows, Wq, Wk, Wv)
    return out


def _pad_docs(ids, mult):
    n = ids.shape[0]
    npad = (-n) % mult
    if npad:
        ids = jnp.concatenate([ids, jnp.zeros((npad,) + ids.shape[1:], ids.dtype)], 0)
    return ids


def kernel(users, items, negs, query_words, query_word_ids, review_word_ids,
           review_entity, purch_src, purch_dst, purch_qid, word_emb, entity_emb,
           Wq, Wk, Wv):
    N = ENTITY_NUM

    rw = _pad_docs(review_word_ids, BLK_ROWS // LR)
    qw_ids = jnp.concatenate([query_word_ids, query_words], 0)
    qw_ids = _pad_docs(qw_ids, BLK_ROWS // LQ)

    xr = word_emb[rw.reshape(-1)]
    xq = word_emb[qw_ids.reshape(-1)]
    review_h = _doc_embed(xr, Wq, Wk, Wv, LR)[:REVIEW_NUM]
    qh_all = _doc_embed(xq, Wq, Wk, Wv, LQ)
    query_h = qh_all[:QUERY_NUM]
    qw = qh_all[QUERY_NUM:QUERY_NUM + B]

    cnt_p = jax.ops.segment_sum(jnp.ones((REVIEW_NUM,), jnp.float32), review_entity,
                                num_segments=N)
    cnt_i = (jax.ops.segment_sum(jnp.ones((EP,), jnp.float32), purch_src, num_segments=N)
             + jax.ops.segment_sum(jnp.ones((EP,), jnp.float32), purch_dst,
                                   num_segments=N))
    invdegp = 1.0 / jnp.maximum(cnt_p, 1.0)
    rsq = 1.0 / jnp.sqrt(jnp.maximum(cnt_i, 1.0))

    ent_h = jax.ops.segment_sum(review_h, review_entity, num_segments=N) * invdegp[:, None]
    e0 = jnp.concatenate([entity_emb, ent_h], -1)
    QE = jax.ops.segment_sum(query_h[purch_qid] * rsq[purch_src][:, None], purch_dst,
                             num_segments=N)

    e, es = e0, e0 * rsq[:, None]
    layers = [e0]
    for _ in range(CONV):
        S = (jax.ops.segment_sum(e[purch_src], purch_dst, num_segments=N)
             + jax.ops.segment_sum(es[purch_dst], purch_src, num_segments=N))
        S = S.at[:, DE:].add(QE)
        e = S * rsq[:, None]
        es = e * rsq[:, None]
        layers.append(e)
    ent_e = (layers[0] + layers[1] + layers[2]) / 3.0

    user_e, item_e, neg_e = ent_e[users], ent_e[items], ent_e[negs]
    pm = user_e + jnp.concatenate([jnp.zeros((B, DE), jnp.float32), qw], -1)
    pos = jnp.sum(pm * item_e, -1)
    neg = jnp.sum(pm * neg_e, -1)
    return -jnp.mean(jax.nn.log_sigmoid(pos - neg))


# speedup vs baseline: 1.3941x; 1.3941x over previous
import functools
import math

import jax
import jax.numpy as jnp
from jax import lax
from jax.experimental import pallas as pl
from jax.experimental.pallas import tpu as pltpu
from jax.experimental.pallas import tpu_sc as plsc

WORD_NUM = 50000
QUERY_NUM = 10000
ENTITY_NUM = 50000
REVIEW_NUM = 25000
DW = 128
DE = 128
HEADS = 4
CONV = 2
LQ = 8
LR = 16
EP = 200000
B = 1024
DH = DW // HEADS

BLK_ROWS = 1024
GRP = 128

NC, NS = 2, 16
NW = NC * NS
SC_MESH = dict(core_axis_name="c", subcore_axis_name="s")
SC_PARAMS = pltpu.CompilerParams(use_tc_tiling_on_sc=False)


def _wid():
    return lax.axis_index("s") * NC + lax.axis_index("c")


GCH = 640


def _sc_gather_rows(table, idx, d):
    n = idx.shape[0]
    per_w = n // NW
    n_it = per_w // GCH
    assert per_w % GCH == 0 and per_w % 8 == 0

    @functools.partial(
        pl.kernel,
        out_type=jax.ShapeDtypeStruct((n, d), jnp.float32),
        mesh=plsc.VectorSubcoreMesh(**SC_MESH),
        scratch_types=[
            pltpu.VMEM((GCH,), jnp.int32),
            pltpu.VMEM((GCH, d), jnp.float32),
            pltpu.SemaphoreType.DMA,
        ],
    )
    def k(table_hbm, idx_hbm, out_hbm, idx_v, rows_v, sem):
        w = _wid()

        def body(it, _):
            base = pl.multiple_of(w * per_w + it * GCH, 8)
            pltpu.sync_copy(idx_hbm.at[pl.ds(base, GCH)], idx_v)
            pltpu.async_copy(table_hbm.at[idx_v], rows_v, sem).wait()
            pltpu.sync_copy(rows_v, out_hbm.at[pl.ds(base, GCH)])
            return 0

        lax.fori_loop(0, n_it, body, 0)

    return k(table, idx)


N_PAD = 51200
EP_PAD = 204800
R_PAD = 25600
Q_PAD = 10016
ROWS_PER_TILE = N_PAD // NS
ECH = 320
DCH = 800


def _v16(j):
    return pl.ds(j * 16, 16)


def _fill_1d(ref, n, value):
    val = jnp.full((16,), value, ref.dtype)

    def body(j, _):
        ref[_v16(j)] = val
        return 0

    lax.fori_loop(0, n // 16, body, 0)


def _newton_rsqrt(x):
    i = lax.bitcast_convert_type(x, jnp.int32)
    i = 0x5F3759DF - lax.shift_right_arithmetic(i, 1)
    y = lax.bitcast_convert_type(i, jnp.float32)
    for _ in range(4):
        y = y * (1.5 - 0.5 * x * y * y)
    return y


def _fill_zeros2d(zeros_v):
    def zb(i, _):
        zeros_v[i, pl.ds(0, 16)] = jnp.zeros((16,), jnp.float32)
        zeros_v[i, pl.ds(16, 16)] = jnp.zeros((16,), jnp.float32)
        return 0

    lax.fori_loop(0, ECH, zb, 0)


def _zero_rows(ref, zeros_v, t):
    def body(z, _):
        pltpu.sync_copy(zeros_v, ref.at[pl.ds(t * ROWS_PER_TILE + z * ECH, ECH)])
        return 0

    lax.fori_loop(0, ROWS_PER_TILE // ECH, body, 0)


def _sc_degrees(src, dst, rev_ent):
    e_per_tile = EP_PAD // NS
    r_per_tile = R_PAD // NS

    @functools.partial(
        pl.kernel,
        out_type=(
            jax.ShapeDtypeStruct((N_PAD,), jnp.float32),
            jax.ShapeDtypeStruct((N_PAD,), jnp.float32),
            jax.ShapeDtypeStruct((N_PAD,), jnp.float32),
        ),
        mesh=plsc.VectorSubcoreMesh(**SC_MESH),
        scratch_types=[
            pltpu.VMEM((DCH,), jnp.int32),
            pltpu.VMEM((DCH,), jnp.float32),
            pltpu.VMEM((ROWS_PER_TILE,), jnp.float32),
            pltpu.VMEM((ROWS_PER_TILE,), jnp.float32),
            pltpu.VMEM_SHARED((N_PAD,), jnp.float32),
        ],
    )
    def k(src_hbm, dst_hbm, re_hbm, rsq_hbm, rsq2_hbm, invdp_hbm,
          idx_v, ones_v, buf_v, buf2_v, S):
        c = lax.axis_index("c")
        t = lax.axis_index("s")
        _fill_1d(ones_v, DCH, 1.0)
        _fill_1d(buf_v, ROWS_PER_TILE, 0.0)
        pltpu.sync_copy(buf_v, S.at[pl.ds(t * ROWS_PER_TILE, ROWS_PER_TILE)])
        plsc.subcore_barrier()

        def edge_pass(ref):
            def body(it, _):
                base = pl.multiple_of(t * e_per_tile + it * DCH, 8)
                pltpu.sync_copy(ref.at[pl.ds(base, DCH)], idx_v)
                pltpu.sync_copy(ones_v, S.at[idx_v], add=True)
                return 0

            lax.fori_loop(0, e_per_tile // DCH, body, 0)

        @pl.when(c == 0)
        def _():
            edge_pass(src_hbm)
            edge_pass(dst_hbm)

        @pl.when(c == 1)
        def _():
            def body(it, _):
                base = pl.multiple_of(t * r_per_tile + it * DCH, 8)
                pltpu.sync_copy(re_hbm.at[pl.ds(base, DCH)], idx_v)
                pltpu.sync_copy(ones_v, S.at[idx_v], add=True)
                return 0

            lax.fori_loop(0, r_per_tile // DCH, body, 0)

        plsc.subcore_barrier()
        row0 = t * ROWS_PER_TILE
        pltpu.sync_copy(S.at[pl.ds(row0, ROWS_PER_TILE)], buf_v)

        def out_body(j, _):
            v = jnp.maximum(buf_v[_v16(j)], 1.0)
            r = _newton_rsqrt(v)
            buf_v[_v16(j)] = r
            buf2_v[_v16(j)] = r * r
            return 0

        lax.fori_loop(0, ROWS_PER_TILE // 16, out_body, 0)

        @pl.when(c == 0)
        def _():
            pltpu.sync_copy(buf_v, rsq_hbm.at[pl.ds(row0, ROWS_PER_TILE)])
            pltpu.sync_copy(buf2_v, rsq2_hbm.at[pl.ds(row0, ROWS_PER_TILE)])

        @pl.when(c == 1)
        def _():
            pltpu.sync_copy(buf2_v, invdp_hbm.at[pl.ds(row0, ROWS_PER_TILE)])

    return k(src, dst, rev_ent)


def _mul_rows(dst_v, w_v, n):
    def body(i, _):
        dst_v[i, pl.ds(0, 16)] = dst_v[i, pl.ds(0, 16)] * w_v[i, pl.ds(0, 16)]
        dst_v[i, pl.ds(16, 16)] = dst_v[i, pl.ds(16, 16)] * w_v[i, pl.ds(16, 16)]
        return 0

    lax.fori_loop(0, n, body, 0)


def _sc_graph_init(rh_parts, qh_parts, qid, src, dst, re, rsq32):
    r_per_tile = R_PAD // NS
    e_per_tile = EP_PAD // NS
    out_t = tuple(jax.ShapeDtypeStruct((N_PAD, 32), jnp.float32)
                  for _ in range(8))

    @functools.partial(
        pl.kernel,
        out_type=out_t,
        mesh=plsc.VectorSubcoreMesh(**SC_MESH),
        compiler_params=SC_PARAMS,
        scratch_types=[
            pltpu.VMEM((ECH,), jnp.int32),
            pltpu.VMEM((ECH,), jnp.int32),
            pltpu.VMEM((ECH,), jnp.int32),
            pltpu.VMEM((ECH,), jnp.int32),
            pltpu.VMEM((ECH, 32), jnp.float32),
            pltpu.VMEM((ECH, 32), jnp.float32),
            pltpu.VMEM_SHARED((N_PAD, 32), jnp.float32),
            pltpu.SemaphoreType.DMA,
            pltpu.SemaphoreType.DMA,
        ],
    )
    def k(rh0, rh1, rh2, rh3, qh0, qh1, qh2, qh3,
          qid_hbm, src_hbm, dst_hbm, re_hbm, rsq32_hbm,
          r0, r1, r2, r3, q0, q1, q2, q3,
          ia_v, ib_v, ic_v, id_v, rows_v, w_v, S, sem, sem2):
        rh = [rh0, rh1, rh2, rh3]
        qh = [qh0, qh1, qh2, qh3]
        rev_out = [r0, r1, r2, r3]
        qe_out = [q0, q1, q2, q3]
        c = lax.axis_index("c")
        t = lax.axis_index("s")
        tile_rows = pl.ds(t * ROWS_PER_TILE, ROWS_PER_TILE)

        def do_part(pp):
            _fill_zeros2d(w_v)
            _zero_rows(S, w_v, t)
            plsc.subcore_barrier()

            def rev_body(it, _):
                base = pl.multiple_of(t * r_per_tile + it * ECH, 8)
                pltpu.sync_copy(re_hbm.at[pl.ds(base, ECH)], ia_v)
                pltpu.sync_copy(rh[pp].at[pl.ds(base, ECH)], rows_v)
                pltpu.sync_copy(rows_v, S.at[ia_v], add=True)
                return 0

            lax.fori_loop(0, r_per_tile // ECH, rev_body, 0)
            plsc.subcore_barrier()
            pltpu.sync_copy(S.at[tile_rows], rev_out[pp].at[tile_rows])
            plsc.subcore_barrier()

            _zero_rows(S, w_v, t)
            plsc.subcore_barrier()

            def q_body(it, _):
                base = pl.multiple_of(t * e_per_tile + it * ECH, 8)
                pltpu.sync_copy(qid_hbm.at[pl.ds(base, ECH)], ib_v)
                pltpu.sync_copy(src_hbm.at[pl.ds(base, ECH)], ic_v)
                pltpu.sync_copy(dst_hbm.at[pl.ds(base, ECH)], id_v)
                cp1 = pltpu.async_copy(qh[pp].at[ib_v], rows_v, sem)
                cp2 = pltpu.async_copy(rsq32_hbm.at[ic_v], w_v, sem2)
                cp1.wait()
                cp2.wait()
                _mul_rows(rows_v, w_v, ECH)
                pltpu.sync_copy(rows_v, S.at[id_v], add=True)
                return 0

            lax.fori_loop(0, e_per_tile // ECH, q_body, 0)
            plsc.subcore_barrier()
            pltpu.sync_copy(S.at[tile_rows], qe_out[pp].at[tile_rows])
            plsc.subcore_barrier()

        @pl.when(c == 0)
        def _():
            do_part(0)
            do_part(1)

        @pl.when(c == 1)
        def _():
            do_part(2)
            do_part(3)

    outs = k(*rh_parts, *qh_parts, qid, src, dst, re, rsq32)
    return list(outs[0:4]), list(outs[4:8])


def _sc_conv_layer(e_parts, es_parts, qe_parts, src, dst):
    e_per_tile = EP_PAD // NS
    out_t = tuple(jax.ShapeDtypeStruct((N_PAD, 32), jnp.float32)
                  for _ in range(8))

    @functools.partial(
        pl.kernel,
        out_type=out_t,
        mesh=plsc.VectorSubcoreMesh(**SC_MESH),
        compiler_params=SC_PARAMS,
        scratch_types=[
            pltpu.VMEM((ECH,), jnp.int32),
            pltpu.VMEM((ECH,), jnp.int32),
            pltpu.VMEM((ECH, 32), jnp.float32),
            pltpu.VMEM((ECH, 32), jnp.float32),
            pltpu.VMEM_SHARED((N_PAD, 32), jnp.float32),
            pltpu.SemaphoreType.DMA,
            pltpu.SemaphoreType.DMA,
        ],
    )
    def k(*args):
        e_in = args[0:8]
        es_in = args[8:16]
        qe_in = args[16:20]
        src_hbm, dst_hbm = args[20:22]
        outs = args[22:30]
        src_v, dst_v, rows_v, rows2_v, S, sem, sem2 = args[30:]
        c = lax.axis_index("c")
        t = lax.axis_index("s")
        tile_rows = pl.ds(t * ROWS_PER_TILE, ROWS_PER_TILE)

        def do_part(P):
            if P >= 4:
                pltpu.sync_copy(qe_in[P - 4].at[tile_rows], S.at[tile_rows])
            else:
                _fill_zeros2d(rows2_v)
                _zero_rows(S, rows2_v, t)
            plsc.subcore_barrier()

            def edge_body(it, _):
                base = pl.multiple_of(t * e_per_tile + it * ECH, 8)
                pltpu.sync_copy(src_hbm.at[pl.ds(base, ECH)], src_v)
                pltpu.sync_copy(dst_hbm.at[pl.ds(base, ECH)], dst_v)
                cp1 = pltpu.async_copy(e_in[P].at[src_v], rows_v, sem)
                cp2 = pltpu.async_copy(es_in[P].at[dst_v], rows2_v, sem2)
                cp1.wait()
                pltpu.sync_copy(rows_v, S.at[dst_v], add=True)
                cp2.wait()
                pltpu.sync_copy(rows2_v, S.at[src_v], add=True)
                return 0

            lax.fori_loop(0, e_per_tile // ECH, edge_body, 0)
            plsc.subcore_barrier()
            pltpu.sync_copy(S.at[tile_rows], outs[P].at[tile_rows])
            plsc.subcore_barrier()

        @pl.when(c == 0)
        def _():
            for P in range(4):
                do_part(P)

        @pl.when(c == 1)
        def _():
            for P in range(4, 8):
                do_part(P)

    outs = k(*e_parts, *es_parts, *qe_parts, src, dst)
    return list(outs)


def _sc_final_gather(layer_parts, users, items, negs):
    tabs = [p for lp in layer_parts for p in lp]
    out_t = tuple(jax.ShapeDtypeStruct((8, B, 32), jnp.float32) for _ in range(3))

    @functools.partial(
        pl.kernel,
        out_type=out_t,
        mesh=plsc.VectorSubcoreMesh(**SC_MESH),
        compiler_params=SC_PARAMS,
        scratch_types=[
            pltpu.VMEM((B,), jnp.int32),
            pltpu.VMEM((B, 32), jnp.float32),
            pltpu.VMEM((B, 32), jnp.float32),
            pltpu.VMEM((B, 32), jnp.float32),
            pltpu.SemaphoreType.DMA,
        ],
    )
    def k(*args):
        tab = args[0:24]
        u_hbm, i_hbm, n_hbm = args[24:27]
        outs = args[27:30]
        idx_v, b0, b1, b2, sem = args[30:]
        w = _wid()
        idx_srcs = [u_hbm, i_hbm, n_hbm]
        third = jnp.full((16,), 1.0 / 3.0, jnp.float32)
        for task in range(24):
            st, P = divmod(task, 8)

            @pl.when(w == task)
            def _():
                pltpu.sync_copy(idx_srcs[st], idx_v)
                pltpu.async_copy(tab[0 * 8 + P].at[idx_v], b0, sem).wait()
                pltpu.async_copy(tab[1 * 8 + P].at[idx_v], b1, sem).wait()
                pltpu.async_copy(tab[2 * 8 + P].at[idx_v], b2, sem).wait()

                def sum_b(i, _):
                    for cs in (0, 16):
                        v = (b0[i, pl.ds(cs, 16)] + b1[i, pl.ds(cs, 16)]
                             + b2[i, pl.ds(cs, 16)]) * third
                        b0[i, pl.ds(cs, 16)] = v
                    return 0

                lax.fori_loop(0, B, sum_b, 0)
                pltpu.sync_copy(b0, outs[st].at[P])

    return k(*tabs, users, items, negs)


BRS = 512


def _prep_body(rsq_ref, invdp_ref, ir_ref, rsq32_ref):
    r = rsq_ref[...]
    ir_ref[...] = r * invdp_ref[...]
    rsq32_ref[...] = jnp.broadcast_to(r, (BRS, 32))


def _tc_prep(rsq2d, invdp2d):
    return pl.pallas_call(
        _prep_body,
        grid=(N_PAD // BRS,),
        in_specs=[
            pl.BlockSpec((BRS, 1), lambda i: (i, 0)),
            pl.BlockSpec((BRS, 1), lambda i: (i, 0)),
        ],
        out_specs=[
            pl.BlockSpec((BRS, 1), lambda i: (i, 0)),
            pl.BlockSpec((BRS, 32), lambda i: (i, 0)),
        ],
        out_shape=[
            jax.ShapeDtypeStruct((N_PAD, 1), jnp.float32),
            jax.ShapeDtypeStruct((N_PAD, 32), jnp.float32),
        ],
    )(rsq2d, invdp2d)


def _scale_body(n_parts, n_scales, sidx, *refs):
    ins = refs[:n_parts]
    scs = refs[n_parts:n_parts + n_scales]
    outs = refs[n_parts + n_scales:]
    for j in range(n_parts):
        outs[j][...] = ins[j][...] * scs[sidx[j]][...]


def _tc_scale_parts(parts, scales, sidx):
    n_parts, n_scales = len(parts), len(scales)
    out = pl.pallas_call(
        functools.partial(_scale_body, n_parts, n_scales, tuple(sidx)),
        grid=(N_PAD // BRS,),
        in_specs=[pl.BlockSpec((BRS, 32), lambda i: (i, 0))] * n_parts
        + [pl.BlockSpec((BRS, 1), lambda i: (i, 0))] * n_scales,
        out_specs=[pl.BlockSpec((BRS, 32), lambda i: (i, 0))] * n_parts,
        out_shape=[jax.ShapeDtypeStruct((N_PAD, 32), jnp.float32)] * n_parts,
    )(*parts, *scales)
    return list(out)


def _loss_body(u_ref, i_ref, n_ref, qw_ref, o_ref):
    pos = jnp.zeros((B, 1), jnp.float32)
    neg = jnp.zeros((B, 1), jnp.float32)
    for P in range(8):
        pm = u_ref[P]
        if P >= 4:
            pm = pm + qw_ref[:, (P - 4) * 32:(P - 3) * 32]
        pos = pos + jnp.sum(pm * i_ref[P], axis=1, keepdims=True)
        neg = neg + jnp.sum(pm * n_ref[P], axis=1, keepdims=True)
    x = pos - neg
    ls = jnp.minimum(x, 0.0) - jnp.log1p(jnp.exp(-jnp.abs(x)))
    o_ref[...] = jnp.reshape(-jnp.sum(ls) / B, (1, 1))


def _tc_loss(u_e, i_e, n_e, qw):
    out = pl.pallas_call(
        _loss_body,
        out_shape=jax.ShapeDtypeStruct((1, 1), jnp.float32),
    )(u_e, i_e, n_e, qw)
    return out[0, 0]


def _doc_embed_body(L, x_ref, wq_ref, wk_ref, wv_ref, o_ref, o_scratch):
    bf = jnp.bfloat16
    x = x_ref[...].astype(bf)
    q = jnp.dot(x, wq_ref[...].astype(bf), preferred_element_type=jnp.float32)
    k = jnp.dot(x, wk_ref[...].astype(bf), preferred_element_type=jnp.float32)
    v = jnp.dot(x, wv_ref[...].astype(bf), preferred_element_type=jnp.float32)
    scale = 1.0 / math.sqrt(DH)
    ri = lax.broadcasted_iota(jnp.int32, (GRP, GRP), 0) // L
    ci = lax.broadcasted_iota(jnp.int32, (GRP, GRP), 1) // L
    mask = ri == ci
    n_grp = BLK_ROWS // GRP
    for g in range(n_grp):
        qg = q[g * GRP:(g + 1) * GRP, :]
        kg = k[g * GRP:(g + 1) * GRP, :]
        vg = v[g * GRP:(g + 1) * GRP, :]
        for h in range(HEADS):
            qh = (qg[:, h * DH:(h + 1) * DH] * scale).astype(bf)
            kh = kg[:, h * DH:(h + 1) * DH].astype(bf)
            vh = vg[:, h * DH:(h + 1) * DH].astype(bf)
            s = lax.dot_general(qh, kh, (((1,), (1,)), ((), ())),
                                preferred_element_type=jnp.float32)
            s = jnp.where(mask, s, -1e30)
            s = s - jnp.max(s, axis=-1, keepdims=True)
            p = jnp.exp(s)
            p = (p / jnp.sum(p, axis=-1, keepdims=True)).astype(bf)
            oh = jnp.dot(p, vh, preferred_element_type=jnp.float32)
            o_scratch[g * GRP:(g + 1) * GRP, h * DH:(h + 1) * DH] = oh.astype(bf)
    n_docs = BLK_ROWS // L
    pr = lax.broadcasted_iota(jnp.int32, (n_docs, BLK_ROWS), 0)
    pc = lax.broadcasted_iota(jnp.int32, (n_docs, BLK_ROWS), 1) // L
    pool = jnp.where(pr == pc, 1.0 / L, 0.0).astype(bf)
    o_ref[...] = jnp.dot(pool, o_scratch[...], preferred_element_type=jnp.float32)


def _doc_embed(xrows, Wq, Wk, Wv, L, blk_off, n_blocks):
    docs_per_blk = BLK_ROWS // L
    out = pl.pallas_call(
        functools.partial(_doc_embed_body, L),
        grid=(n_blocks,),
        in_specs=[
            pl.BlockSpec((BLK_ROWS, DW), lambda i: (i + blk_off, 0)),
            pl.BlockSpec((DW, DW), lambda i: (0, 0)),
            pl.BlockSpec((DW, DW), lambda i: (0, 0)),
            pl.BlockSpec((DW, DW), lambda i: (0, 0)),
        ],
        out_specs=pl.BlockSpec((docs_per_blk, DW), lambda i: (i, 0)),
        out_shape=jax.ShapeDtypeStruct((n_blocks * docs_per_blk, DW), jnp.float32),
        scratch_shapes=[pltpu.VMEM((BLK_ROWS, DW), jnp.bfloat16)],
    )(xrows, Wq, Wk, Wv)
    return out


def _pad_docs(ids, mult):
    n = ids.shape[0]
    npad = (-n) % mult
    if npad:
        ids = jnp.concatenate([ids, jnp.zeros((npad,) + ids.shape[1:], ids.dtype)], 0)
    return ids


def kernel(users, items, negs, query_words, query_word_ids, review_word_ids,
           review_entity, purch_src, purch_dst, purch_qid, word_emb, entity_emb,
           Wq, Wk, Wv):
    N = ENTITY_NUM

    rw = _pad_docs(review_word_ids, BLK_ROWS // LR)
    qw_ids = jnp.concatenate([query_word_ids, query_words], 0)
    qw_ids = _pad_docs(qw_ids, BLK_ROWS // LQ)
    r_blocks = rw.shape[0] * LR // BLK_ROWS
    q_blocks = qw_ids.shape[0] * LQ // BLK_ROWS

    idx_all = jnp.concatenate([rw.reshape(-1), qw_ids.reshape(-1)])
    npad = (-idx_all.shape[0]) % (NW * GCH)
    idx_all = jnp.concatenate([idx_all, jnp.zeros((npad,), idx_all.dtype)])
    xrows = _sc_gather_rows(word_emb, idx_all.astype(jnp.int32), DW)

    review_h = _doc_embed(xrows, Wq, Wk, Wv, LR, 0, r_blocks)[:REVIEW_NUM]
    qh_all = _doc_embed(xrows, Wq, Wk, Wv, LQ, r_blocks, q_blocks)
    query_h = qh_all[:QUERY_NUM]
    qw = qh_all[QUERY_NUM:QUERY_NUM + B]

    def pad_idx(a, n, fill):
        return jnp.concatenate(
            [a.astype(jnp.int32), jnp.full((n - a.shape[0],), fill, jnp.int32)])

    src_p = pad_idx(purch_src, EP_PAD, ENTITY_NUM)
    dst_p = pad_idx(purch_dst, EP_PAD, ENTITY_NUM)
    re_p = pad_idx(review_entity, R_PAD, ENTITY_NUM)
    rsq_f, rsq2_f, invdp_f = _sc_degrees(src_p, dst_p, re_p)
    rsq2d = rsq_f[:, None]
    rsq2_2d = rsq2_f[:, None]
    invdp2d = invdp_f[:, None]
    invdp_rsq2d, rsq32 = _tc_prep(rsq2d, invdp2d)

    qid_p = pad_idx(purch_qid, EP_PAD, QUERY_NUM)
    rh_pad = jnp.concatenate(
        [review_h, jnp.zeros((R_PAD - review_h.shape[0], DW), jnp.float32)], 0)
    qh_pad = jnp.concatenate(
        [query_h, jnp.zeros((Q_PAD - QUERY_NUM, DW), jnp.float32)], 0)
    ent_pad = jnp.concatenate(
        [entity_emb, jnp.zeros((N_PAD - N, DE), jnp.float32)], 0)
    rh_parts = [rh_pad[:, 32 * j:32 * j + 32] for j in range(4)]
    qh_parts = [qh_pad[:, 32 * j:32 * j + 32] for j in range(4)]
    ent_parts = [ent_pad[:, 32 * j:32 * j + 32] for j in range(4)]

    raw47, qe_p = _sc_graph_init(rh_parts, qh_parts, qid_p, src_p, dst_p,
                                 re_p, rsq32)

    outs = _tc_scale_parts(
        ent_parts + raw47 + raw47,
        [rsq2d, invdp2d, invdp_rsq2d],
        [0] * 4 + [1] * 4 + [2] * 4)
    es0_parts = outs[0:4] + outs[8:12]
    e0_parts = ent_parts + outs[4:8]

    s1 = _sc_conv_layer(e0_parts, es0_parts, qe_p, src_p, dst_p)
    outs = _tc_scale_parts(s1 + s1, [rsq2d, rsq2_2d], [0] * 8 + [1] * 8)
    e1_parts = outs[0:8]
    es1_parts = outs[8:16]
    s2 = _sc_conv_layer(e1_parts, es1_parts, qe_p, src_p, dst_p)
    e2_parts = _tc_scale_parts(s2, [rsq2d], [0] * 8)

    u_e, i_e, n_e = _sc_final_gather(
        [e0_parts, e1_parts, e2_parts],
        users.astype(jnp.int32), items.astype(jnp.int32), negs.astype(jnp.int32))
    return _tc_loss(u_e, i_e, n_e, qw)


# speedup vs baseline: 1.4049x; 1.0077x over previous
import functools
import math

import jax
import jax.numpy as jnp
from jax import lax
from jax.experimental import pallas as pl
from jax.experimental.pallas import tpu as pltpu
from jax.experimental.pallas import tpu_sc as plsc

WORD_NUM = 50000
QUERY_NUM = 10000
ENTITY_NUM = 50000
REVIEW_NUM = 25000
DW = 128
DE = 128
HEADS = 4
CONV = 2
LQ = 8
LR = 16
EP = 200000
B = 1024
DH = DW // HEADS

BLK_ROWS = 1024
GRP = 128

NC, NS = 2, 16
NW = NC * NS
SC_MESH = dict(core_axis_name="c", subcore_axis_name="s")
SC_PARAMS = pltpu.CompilerParams(use_tc_tiling_on_sc=False)


def _wid():
    return lax.axis_index("s") * NC + lax.axis_index("c")


GCH = 640


def _sc_gather_rows(table, idx, d):
    n = idx.shape[0]
    per_w = n // NW
    n_it = per_w // GCH
    assert per_w % GCH == 0 and per_w % 8 == 0

    @functools.partial(
        pl.kernel,
        out_type=jax.ShapeDtypeStruct((n, d), jnp.float32),
        mesh=plsc.VectorSubcoreMesh(**SC_MESH),
        scratch_types=[
            pltpu.VMEM((GCH,), jnp.int32),
            pltpu.VMEM((GCH, d), jnp.float32),
            pltpu.SemaphoreType.DMA,
        ],
    )
    def k(table_hbm, idx_hbm, out_hbm, idx_v, rows_v, sem):
        w = _wid()

        def body(it, _):
            base = pl.multiple_of(w * per_w + it * GCH, 8)
            pltpu.sync_copy(idx_hbm.at[pl.ds(base, GCH)], idx_v)
            pltpu.async_copy(table_hbm.at[idx_v], rows_v, sem).wait()
            pltpu.sync_copy(rows_v, out_hbm.at[pl.ds(base, GCH)])
            return 0

        lax.fori_loop(0, n_it, body, 0)

    return k(table, idx)


N_PAD = 51200
EP_PAD = 204800
R_PAD = 25600
Q_PAD = 10016
ROWS_PER_TILE = N_PAD // NS
ECH = 400
DCH = 800


def _v16(j):
    return pl.ds(j * 16, 16)


def _fill_1d(ref, n, value):
    val = jnp.full((16,), value, ref.dtype)

    def body(j, _):
        ref[_v16(j)] = val
        return 0

    lax.fori_loop(0, n // 16, body, 0)


def _newton_rsqrt(x):
    i = lax.bitcast_convert_type(x, jnp.int32)
    i = 0x5F3759DF - lax.shift_right_arithmetic(i, 1)
    y = lax.bitcast_convert_type(i, jnp.float32)
    for _ in range(4):
        y = y * (1.5 - 0.5 * x * y * y)
    return y


def _fill_zeros2d(zeros_v):
    def zb(i, _):
        zeros_v[i, pl.ds(0, 16)] = jnp.zeros((16,), jnp.float32)
        zeros_v[i, pl.ds(16, 16)] = jnp.zeros((16,), jnp.float32)
        return 0

    lax.fori_loop(0, ECH, zb, 0)


def _zero_rows(ref, zeros_v, t):
    def body(z, _):
        pltpu.sync_copy(zeros_v, ref.at[pl.ds(t * ROWS_PER_TILE + z * ECH, ECH)])
        return 0

    lax.fori_loop(0, ROWS_PER_TILE // ECH, body, 0)


def _sc_degrees(src, dst, rev_ent):
    e_per_tile = EP_PAD // NS
    r_per_tile = R_PAD // NS

    @functools.partial(
        pl.kernel,
        out_type=(
            jax.ShapeDtypeStruct((N_PAD,), jnp.float32),
            jax.ShapeDtypeStruct((N_PAD,), jnp.float32),
            jax.ShapeDtypeStruct((N_PAD,), jnp.float32),
        ),
        mesh=plsc.VectorSubcoreMesh(**SC_MESH),
        scratch_types=[
            pltpu.VMEM((DCH,), jnp.int32),
            pltpu.VMEM((DCH,), jnp.float32),
            pltpu.VMEM((ROWS_PER_TILE,), jnp.float32),
            pltpu.VMEM((ROWS_PER_TILE,), jnp.float32),
            pltpu.VMEM_SHARED((N_PAD,), jnp.float32),
        ],
    )
    def k(src_hbm, dst_hbm, re_hbm, rsq_hbm, rsq2_hbm, invdp_hbm,
          idx_v, ones_v, buf_v, buf2_v, S):
        c = lax.axis_index("c")
        t = lax.axis_index("s")
        _fill_1d(ones_v, DCH, 1.0)
        _fill_1d(buf_v, ROWS_PER_TILE, 0.0)
        pltpu.sync_copy(buf_v, S.at[pl.ds(t * ROWS_PER_TILE, ROWS_PER_TILE)])
        plsc.subcore_barrier()

        def edge_pass(ref):
            def body(it, _):
                base = pl.multiple_of(t * e_per_tile + it * DCH, 8)
                pltpu.sync_copy(ref.at[pl.ds(base, DCH)], idx_v)
                pltpu.sync_copy(ones_v, S.at[idx_v], add=True)
                return 0

            lax.fori_loop(0, e_per_tile // DCH, body, 0)

        @pl.when(c == 0)
        def _():
            edge_pass(src_hbm)
            edge_pass(dst_hbm)

        @pl.when(c == 1)
        def _():
            def body(it, _):
                base = pl.multiple_of(t * r_per_tile + it * DCH, 8)
                pltpu.sync_copy(re_hbm.at[pl.ds(base, DCH)], idx_v)
                pltpu.sync_copy(ones_v, S.at[idx_v], add=True)
                return 0

            lax.fori_loop(0, r_per_tile // DCH, body, 0)

        plsc.subcore_barrier()
        row0 = t * ROWS_PER_TILE
        pltpu.sync_copy(S.at[pl.ds(row0, ROWS_PER_TILE)], buf_v)

        def out_body(j, _):
            v = jnp.maximum(buf_v[_v16(j)], 1.0)
            r = _newton_rsqrt(v)
            buf_v[_v16(j)] = r
            buf2_v[_v16(j)] = r * r
            return 0

        lax.fori_loop(0, ROWS_PER_TILE // 16, out_body, 0)

        @pl.when(c == 0)
        def _():
            pltpu.sync_copy(buf_v, rsq_hbm.at[pl.ds(row0, ROWS_PER_TILE)])
            pltpu.sync_copy(buf2_v, rsq2_hbm.at[pl.ds(row0, ROWS_PER_TILE)])

        @pl.when(c == 1)
        def _():
            pltpu.sync_copy(buf2_v, invdp_hbm.at[pl.ds(row0, ROWS_PER_TILE)])

    return k(src, dst, rev_ent)


def _mul_rows(dst_v, w_v, n):
    def body(i, _):
        dst_v[i, pl.ds(0, 16)] = dst_v[i, pl.ds(0, 16)] * w_v[i, pl.ds(0, 16)]
        dst_v[i, pl.ds(16, 16)] = dst_v[i, pl.ds(16, 16)] * w_v[i, pl.ds(16, 16)]
        return 0

    lax.fori_loop(0, n, body, 0)


def _sc_graph_init(rh_parts, qh_parts, qid, src, dst, re, rsq32):
    r_per_tile = R_PAD // NS
    e_per_tile = EP_PAD // NS
    out_t = tuple(jax.ShapeDtypeStruct((N_PAD, 32), jnp.float32)
                  for _ in range(8))

    @functools.partial(
        pl.kernel,
        out_type=out_t,
        mesh=plsc.VectorSubcoreMesh(**SC_MESH),
        compiler_params=SC_PARAMS,
        scratch_types=[
            pltpu.VMEM((ECH,), jnp.int32),
            pltpu.VMEM((ECH,), jnp.int32),
            pltpu.VMEM((ECH,), jnp.int32),
            pltpu.VMEM((ECH,), jnp.int32),
            pltpu.VMEM((ECH, 32), jnp.float32),
            pltpu.VMEM((ECH, 32), jnp.float32),
            pltpu.VMEM_SHARED((N_PAD, 32), jnp.float32),
            pltpu.SemaphoreType.DMA,
            pltpu.SemaphoreType.DMA,
        ],
    )
    def k(rh0, rh1, rh2, rh3, qh0, qh1, qh2, qh3,
          qid_hbm, src_hbm, dst_hbm, re_hbm, rsq32_hbm,
          r0, r1, r2, r3, q0, q1, q2, q3,
          ia_v, ib_v, ic_v, id_v, rows_v, w_v, S, sem, sem2):
        rh = [rh0, rh1, rh2, rh3]
        qh = [qh0, qh1, qh2, qh3]
        rev_out = [r0, r1, r2, r3]
        qe_out = [q0, q1, q2, q3]
        c = lax.axis_index("c")
        t = lax.axis_index("s")
        tile_rows = pl.ds(t * ROWS_PER_TILE, ROWS_PER_TILE)

        def do_part(pp):
            _fill_zeros2d(w_v)
            _zero_rows(S, w_v, t)
            plsc.subcore_barrier()

            def rev_body(it, _):
                base = pl.multiple_of(t * r_per_tile + it * ECH, 8)
                pltpu.sync_copy(re_hbm.at[pl.ds(base, ECH)], ia_v)
                pltpu.sync_copy(rh[pp].at[pl.ds(base, ECH)], rows_v)
                pltpu.sync_copy(rows_v, S.at[ia_v], add=True)
                return 0

            lax.fori_loop(0, r_per_tile // ECH, rev_body, 0)
            plsc.subcore_barrier()
            pltpu.sync_copy(S.at[tile_rows], rev_out[pp].at[tile_rows])
            plsc.subcore_barrier()

            _zero_rows(S, w_v, t)
            plsc.subcore_barrier()

            def q_body(it, _):
                base = pl.multiple_of(t * e_per_tile + it * ECH, 8)
                pltpu.sync_copy(qid_hbm.at[pl.ds(base, ECH)], ib_v)
                pltpu.sync_copy(src_hbm.at[pl.ds(base, ECH)], ic_v)
                pltpu.sync_copy(dst_hbm.at[pl.ds(base, ECH)], id_v)
                cp1 = pltpu.async_copy(qh[pp].at[ib_v], rows_v, sem)
                cp2 = pltpu.async_copy(rsq32_hbm.at[ic_v], w_v, sem2)
                cp1.wait()
                cp2.wait()
                _mul_rows(rows_v, w_v, ECH)
                pltpu.sync_copy(rows_v, S.at[id_v], add=True)
                return 0

            lax.fori_loop(0, e_per_tile // ECH, q_body, 0)
            plsc.subcore_barrier()
            pltpu.sync_copy(S.at[tile_rows], qe_out[pp].at[tile_rows])
            plsc.subcore_barrier()

        @pl.when(c == 0)
        def _():
            do_part(0)
            do_part(1)

        @pl.when(c == 1)
        def _():
            do_part(2)
            do_part(3)

    outs = k(*rh_parts, *qh_parts, qid, src, dst, re, rsq32)
    return list(outs[0:4]), list(outs[4:8])


def _sc_conv_layer(e_parts, es_parts, qe_parts, src, dst):
    e_per_tile = EP_PAD // NS
    out_t = tuple(jax.ShapeDtypeStruct((N_PAD, 32), jnp.float32)
                  for _ in range(8))

    @functools.partial(
        pl.kernel,
        out_type=out_t,
        mesh=plsc.VectorSubcoreMesh(**SC_MESH),
        compiler_params=SC_PARAMS,
        scratch_types=[
            pltpu.VMEM((ECH,), jnp.int32),
            pltpu.VMEM((ECH,), jnp.int32),
            pltpu.VMEM((ECH, 32), jnp.float32),
            pltpu.VMEM((ECH, 32), jnp.float32),
            pltpu.VMEM_SHARED((N_PAD, 32), jnp.float32),
            pltpu.SemaphoreType.DMA,
            pltpu.SemaphoreType.DMA,
        ],
    )
    def k(*args):
        e_in = args[0:8]
        es_in = args[8:16]
        qe_in = args[16:20]
        src_hbm, dst_hbm = args[20:22]
        outs = args[22:30]
        src_v, dst_v, rows_v, rows2_v, S, sem, sem2 = args[30:]
        c = lax.axis_index("c")
        t = lax.axis_index("s")
        tile_rows = pl.ds(t * ROWS_PER_TILE, ROWS_PER_TILE)

        def do_part(P):
            if P >= 4:
                pltpu.sync_copy(qe_in[P - 4].at[tile_rows], S.at[tile_rows])
            else:
                _fill_zeros2d(rows2_v)
                _zero_rows(S, rows2_v, t)
            plsc.subcore_barrier()

            def edge_body(it, _):
                base = pl.multiple_of(t * e_per_tile + it * ECH, 8)
                pltpu.sync_copy(src_hbm.at[pl.ds(base, ECH)], src_v)
                pltpu.sync_copy(dst_hbm.at[pl.ds(base, ECH)], dst_v)
                cp1 = pltpu.async_copy(e_in[P].at[src_v], rows_v, sem)
                cp2 = pltpu.async_copy(es_in[P].at[dst_v], rows2_v, sem2)
                cp1.wait()
                pltpu.sync_copy(rows_v, S.at[dst_v], add=True)
                cp2.wait()
                pltpu.sync_copy(rows2_v, S.at[src_v], add=True)
                return 0

            lax.fori_loop(0, e_per_tile // ECH, edge_body, 0)
            plsc.subcore_barrier()
            pltpu.sync_copy(S.at[tile_rows], outs[P].at[tile_rows])
            plsc.subcore_barrier()

        @pl.when(c == 0)
        def _():
            for P in range(4):
                do_part(P)

        @pl.when(c == 1)
        def _():
            for P in range(4, 8):
                do_part(P)

    outs = k(*e_parts, *es_parts, *qe_parts, src, dst)
    return list(outs)


def _sc_final_gather(layer_parts, users, items, negs):
    tabs = [p for lp in layer_parts for p in lp]
    out_t = tuple(jax.ShapeDtypeStruct((8, B, 32), jnp.float32) for _ in range(3))

    @functools.partial(
        pl.kernel,
        out_type=out_t,
        mesh=plsc.VectorSubcoreMesh(**SC_MESH),
        compiler_params=SC_PARAMS,
        scratch_types=[
            pltpu.VMEM((B,), jnp.int32),
            pltpu.VMEM((B, 32), jnp.float32),
            pltpu.VMEM((B, 32), jnp.float32),
            pltpu.VMEM((B, 32), jnp.float32),
            pltpu.SemaphoreType.DMA,
        ],
    )
    def k(*args):
        tab = args[0:24]
        u_hbm, i_hbm, n_hbm = args[24:27]
        outs = args[27:30]
        idx_v, b0, b1, b2, sem = args[30:]
        w = _wid()
        idx_srcs = [u_hbm, i_hbm, n_hbm]
        third = jnp.full((16,), 1.0 / 3.0, jnp.float32)
        for task in range(24):
            st, P = divmod(task, 8)

            @pl.when(w == task)
            def _():
                pltpu.sync_copy(idx_srcs[st], idx_v)
                pltpu.async_copy(tab[0 * 8 + P].at[idx_v], b0, sem).wait()
                pltpu.async_copy(tab[1 * 8 + P].at[idx_v], b1, sem).wait()
                pltpu.async_copy(tab[2 * 8 + P].at[idx_v], b2, sem).wait()

                def sum_b(i, _):
                    for cs in (0, 16):
                        v = (b0[i, pl.ds(cs, 16)] + b1[i, pl.ds(cs, 16)]
                             + b2[i, pl.ds(cs, 16)]) * third
                        b0[i, pl.ds(cs, 16)] = v
                    return 0

                lax.fori_loop(0, B, sum_b, 0)
                pltpu.sync_copy(b0, outs[st].at[P])

    return k(*tabs, users, items, negs)


BRS = 512


def _prep_body(rsq_ref, invdp_ref, ir_ref, rsq32_ref):
    r = rsq_ref[...]
    ir_ref[...] = r * invdp_ref[...]
    rsq32_ref[...] = jnp.broadcast_to(r, (BRS, 32))


def _tc_prep(rsq2d, invdp2d):
    return pl.pallas_call(
        _prep_body,
        grid=(N_PAD // BRS,),
        in_specs=[
            pl.BlockSpec((BRS, 1), lambda i: (i, 0)),
            pl.BlockSpec((BRS, 1), lambda i: (i, 0)),
        ],
        out_specs=[
            pl.BlockSpec((BRS, 1), lambda i: (i, 0)),
            pl.BlockSpec((BRS, 32), lambda i: (i, 0)),
        ],
        out_shape=[
            jax.ShapeDtypeStruct((N_PAD, 1), jnp.float32),
            jax.ShapeDtypeStruct((N_PAD, 32), jnp.float32),
        ],
    )(rsq2d, invdp2d)


def _scale_body(n_parts, n_scales, sidx, *refs):
    ins = refs[:n_parts]
    scs = refs[n_parts:n_parts + n_scales]
    outs = refs[n_parts + n_scales:]
    for j in range(n_parts):
        outs[j][...] = ins[j][...] * scs[sidx[j]][...]


def _tc_scale_parts(parts, scales, sidx):
    n_parts, n_scales = len(parts), len(scales)
    out = pl.pallas_call(
        functools.partial(_scale_body, n_parts, n_scales, tuple(sidx)),
        grid=(N_PAD // BRS,),
        in_specs=[pl.BlockSpec((BRS, 32), lambda i: (i, 0))] * n_parts
        + [pl.BlockSpec((BRS, 1), lambda i: (i, 0))] * n_scales,
        out_specs=[pl.BlockSpec((BRS, 32), lambda i: (i, 0))] * n_parts,
        out_shape=[jax.ShapeDtypeStruct((N_PAD, 32), jnp.float32)] * n_parts,
    )(*parts, *scales)
    return list(out)


def _loss_body(u_ref, i_ref, n_ref, qw_ref, o_ref):
    pos = jnp.zeros((B, 1), jnp.float32)
    neg = jnp.zeros((B, 1), jnp.float32)
    for P in range(8):
        pm = u_ref[P]
        if P >= 4:
            pm = pm + qw_ref[:, (P - 4) * 32:(P - 3) * 32]
        pos = pos + jnp.sum(pm * i_ref[P], axis=1, keepdims=True)
        neg = neg + jnp.sum(pm * n_ref[P], axis=1, keepdims=True)
    x = pos - neg
    ls = jnp.minimum(x, 0.0) - jnp.log1p(jnp.exp(-jnp.abs(x)))
    o_ref[...] = jnp.reshape(-jnp.sum(ls) / B, (1, 1))


def _tc_loss(u_e, i_e, n_e, qw):
    out = pl.pallas_call(
        _loss_body,
        out_shape=jax.ShapeDtypeStruct((1, 1), jnp.float32),
    )(u_e, i_e, n_e, qw)
    return out[0, 0]


def _doc_embed_body(L, x_ref, wq_ref, wk_ref, wv_ref, o_ref, o_scratch):
    bf = jnp.bfloat16
    x = x_ref[...].astype(bf)
    q = jnp.dot(x, wq_ref[...].astype(bf), preferred_element_type=jnp.float32)
    k = jnp.dot(x, wk_ref[...].astype(bf), preferred_element_type=jnp.float32)
    v = jnp.dot(x, wv_ref[...].astype(bf), preferred_element_type=jnp.float32)
    scale = 1.0 / math.sqrt(DH)
    ri = lax.broadcasted_iota(jnp.int32, (GRP, GRP), 0) // L
    ci = lax.broadcasted_iota(jnp.int32, (GRP, GRP), 1) // L
    mask = ri == ci
    n_grp = BLK_ROWS // GRP
    for g in range(n_grp):
        qg = q[g * GRP:(g + 1) * GRP, :]
        kg = k[g * GRP:(g + 1) * GRP, :]
        vg = v[g * GRP:(g + 1) * GRP, :]
        for h in range(HEADS):
            qh = (qg[:, h * DH:(h + 1) * DH] * scale).astype(bf)
            kh = kg[:, h * DH:(h + 1) * DH].astype(bf)
            vh = vg[:, h * DH:(h + 1) * DH].astype(bf)
            s = lax.dot_general(qh, kh, (((1,), (1,)), ((), ())),
                                preferred_element_type=jnp.float32)
            s = jnp.where(mask, s, -1e30)
            s = s - jnp.max(s, axis=-1, keepdims=True)
            p = jnp.exp(s)
            p = (p / jnp.sum(p, axis=-1, keepdims=True)).astype(bf)
            oh = jnp.dot(p, vh, preferred_element_type=jnp.float32)
            o_scratch[g * GRP:(g + 1) * GRP, h * DH:(h + 1) * DH] = oh.astype(bf)
    n_docs = BLK_ROWS // L
    pr = lax.broadcasted_iota(jnp.int32, (n_docs, BLK_ROWS), 0)
    pc = lax.broadcasted_iota(jnp.int32, (n_docs, BLK_ROWS), 1) // L
    pool = jnp.where(pr == pc, 1.0 / L, 0.0).astype(bf)
    o_ref[...] = jnp.dot(pool, o_scratch[...], preferred_element_type=jnp.float32)


def _doc_embed(xrows, Wq, Wk, Wv, L, blk_off, n_blocks):
    docs_per_blk = BLK_ROWS // L
    out = pl.pallas_call(
        functools.partial(_doc_embed_body, L),
        grid=(n_blocks,),
        in_specs=[
            pl.BlockSpec((BLK_ROWS, DW), lambda i: (i + blk_off, 0)),
            pl.BlockSpec((DW, DW), lambda i: (0, 0)),
            pl.BlockSpec((DW, DW), lambda i: (0, 0)),
            pl.BlockSpec((DW, DW), lambda i: (0, 0)),
        ],
        out_specs=pl.BlockSpec((docs_per_blk, DW), lambda i: (i, 0)),
        out_shape=jax.ShapeDtypeStruct((n_blocks * docs_per_blk, DW), jnp.float32),
        scratch_shapes=[pltpu.VMEM((BLK_ROWS, DW), jnp.bfloat16)],
    )(xrows, Wq, Wk, Wv)
    return out


def _pad_docs(ids, mult):
    n = ids.shape[0]
    npad = (-n) % mult
    if npad:
        ids = jnp.concatenate([ids, jnp.zeros((npad,) + ids.shape[1:], ids.dtype)], 0)
    return ids


def kernel(users, items, negs, query_words, query_word_ids, review_word_ids,
           review_entity, purch_src, purch_dst, purch_qid, word_emb, entity_emb,
           Wq, Wk, Wv):
    N = ENTITY_NUM

    rw = _pad_docs(review_word_ids, BLK_ROWS // LR)
    qw_ids = jnp.concatenate([query_word_ids, query_words], 0)
    qw_ids = _pad_docs(qw_ids, BLK_ROWS // LQ)
    r_blocks = rw.shape[0] * LR // BLK_ROWS
    q_blocks = qw_ids.shape[0] * LQ // BLK_ROWS

    idx_all = jnp.concatenate([rw.reshape(-1), qw_ids.reshape(-1)])
    npad = (-idx_all.shape[0]) % (NW * GCH)
    idx_all = jnp.concatenate([idx_all, jnp.zeros((npad,), idx_all.dtype)])
    xrows = _sc_gather_rows(word_emb, idx_all.astype(jnp.int32), DW)

    review_h = _doc_embed(xrows, Wq, Wk, Wv, LR, 0, r_blocks)[:REVIEW_NUM]
    qh_all = _doc_embed(xrows, Wq, Wk, Wv, LQ, r_blocks, q_blocks)
    query_h = qh_all[:QUERY_NUM]
    qw = qh_all[QUERY_NUM:QUERY_NUM + B]

    def pad_idx(a, n, fill):
        return jnp.concatenate(
            [a.astype(jnp.int32), jnp.full((n - a.shape[0],), fill, jnp.int32)])

    src_p = pad_idx(purch_src, EP_PAD, ENTITY_NUM)
    dst_p = pad_idx(purch_dst, EP_PAD, ENTITY_NUM)
    re_p = pad_idx(review_entity, R_PAD, ENTITY_NUM)
    rsq_f, rsq2_f, invdp_f = _sc_degrees(src_p, dst_p, re_p)
    rsq2d = rsq_f[:, None]
    rsq2_2d = rsq2_f[:, None]
    invdp2d = invdp_f[:, None]
    invdp_rsq2d, rsq32 = _tc_prep(rsq2d, invdp2d)

    qid_p = pad_idx(purch_qid, EP_PAD, QUERY_NUM)
    rh_pad = jnp.concatenate(
        [review_h, jnp.zeros((R_PAD - review_h.shape[0], DW), jnp.float32)], 0)
    qh_pad = jnp.concatenate(
        [query_h, jnp.zeros((Q_PAD - QUERY_NUM, DW), jnp.float32)], 0)
    ent_pad = jnp.concatenate(
        [entity_emb, jnp.zeros((N_PAD - N, DE), jnp.float32)], 0)
    rh_parts = [rh_pad[:, 32 * j:32 * j + 32] for j in range(4)]
    qh_parts = [qh_pad[:, 32 * j:32 * j + 32] for j in range(4)]
    ent_parts = [ent_pad[:, 32 * j:32 * j + 32] for j in range(4)]

    raw47, qe_p = _sc_graph_init(rh_parts, qh_parts, qid_p, src_p, dst_p,
                                 re_p, rsq32)

    outs = _tc_scale_parts(
        ent_parts + raw47 + raw47,
        [rsq2d, invdp2d, invdp_rsq2d],
        [0] * 4 + [1] * 4 + [2] * 4)
    es0_parts = outs[0:4] + outs[8:12]
    e0_parts = ent_parts + outs[4:8]

    s1 = _sc_conv_layer(e0_parts, es0_parts, qe_p, src_p, dst_p)
    outs = _tc_scale_parts(s1 + s1, [rsq2d, rsq2_2d], [0] * 8 + [1] * 8)
    e1_parts = outs[0:8]
    es1_parts = outs[8:16]
    s2 = _sc_conv_layer(e1_parts, es1_parts, qe_p, src_p, dst_p)
    e2_parts = _tc_scale_parts(s2, [rsq2d], [0] * 8)

    u_e, i_e, n_e = _sc_final_gather(
        [e0_parts, e1_parts, e2_parts],
        users.astype(jnp.int32), items.astype(jnp.int32), negs.astype(jnp.int32))
    return _tc_loss(u_e, i_e, n_e, qw)
